# baseline (speedup 1.0000x reference)
import jax
import jax.numpy as jnp
from jax import lax
from jax.experimental import pallas as pl
from jax.experimental.pallas import tpu as pltpu
from jax.experimental.pallas import tpu_sc as plsc

D = 128
NC, NS, L = 2, 16, 16
NW = NC * NS
N_NODES = 50000
E_EDGES = 500000
G = 128
NCHUNK = 6
CH = 8448
ACCR = 8576
CPT = CH // NS
ZPT = ACCR // NS
BKAL = 2048
EPAD = 516096
EP = 524288
NBF = 128
SHF = NBF * G
CACC = 50176

EL = 200000
NBD = 56
EW = NBD * G
ELP = NW * EW

_MESH = plsc.VectorSubcoreMesh(
    core_axis_name="c", subcore_axis_name="s", num_cores=NC, num_subcores=NS
)

_f32 = jnp.float32
_i32 = jnp.int32


def _segsum_body(x_hbm, src_hbm, dst_hbm, meta_hbm, out_hbm,
                 ss0, dd0, rb0, rb1, metab, zbuf, acc, gs0, gs1):
    cid = lax.axis_index("c")
    sid = lax.axis_index("s")
    zv = jnp.zeros((L,), _f32)
    pltpu.sync_copy(meta_hbm.at[cid], metab)
    mv = metab[0, pl.ds(0, 16)]

    def zr(r, _):
        for j in range(D // L):
            zbuf[r, pl.ds(j * L, L)] = zv
        return 0

    lax.fori_loop(0, G, zr, 0)
    z0 = sid * ZPT
    c0 = sid * CPT
    for qq in range(NCHUNK // NC):
        q = qq * NC + cid
        for j in range(ZPT // G):
            pltpu.sync_copy(zbuf, acc.at[pl.ds(z0 + j * G, G)])
        rem = ZPT % G
        if rem:
            pltpu.sync_copy(zbuf.at[pl.ds(0, rem)],
                            acc.at[pl.ds(z0 + (ZPT // G) * G, rem)])
        plsc.subcore_barrier()
        soff = mv[qq]
        nsb = mv[8 + qq]
        my_nb = (nsb - sid + (NS - 1)) // NS

        def body(k, _):
            sb = soff + sid + k * NS
            pltpu.sync_copy(src_hbm.at[sb], ss0)
            pltpu.sync_copy(dst_hbm.at[sb], dd0)
            pltpu.async_copy(x_hbm.at[ss0.at[0]], rb0, gs0)
            for j in range(1, 8):
                rb_j, gs_j = (rb0, gs0) if j % 2 == 0 else (rb1, gs1)
                rb_p, gs_p = (rb0, gs0) if j % 2 == 1 else (rb1, gs1)
                pltpu.async_copy(x_hbm.at[ss0.at[j]], rb_j, gs_j)
                pltpu.make_async_copy(
                    x_hbm.at[ss0.at[j - 1]], rb_p, gs_p).wait()
                pltpu.sync_copy(rb_p, acc.at[dd0.at[j - 1]], add=True)
            pltpu.make_async_copy(x_hbm.at[ss0.at[7]], rb1, gs1).wait()
            pltpu.sync_copy(rb1, acc.at[dd0.at[7]], add=True)
            return 0

        lax.fori_loop(0, my_nb, body, 0)
        plsc.subcore_barrier()
        pltpu.sync_copy(acc.at[pl.ds(c0, CPT)], out_hbm.at[q, pl.ds(c0, CPT)])
        plsc.subcore_barrier()


_segsum = pl.kernel(
    _segsum_body,
    out_type=jax.ShapeDtypeStruct((NCHUNK, CH, D), _f32),
    mesh=_MESH,
    scratch_types=[
        pltpu.VMEM((8, G), _i32),
        pltpu.VMEM((8, G), _i32),
        pltpu.VMEM((G, D), _f32),
        pltpu.VMEM((G, D), _f32),
        pltpu.VMEM((8, G), _i32),
        pltpu.VMEM((G, D), _f32),
        pltpu.VMEM_SHARED((ACCR, D), _f32),
        pltpu.SemaphoreType.DMA,
        pltpu.SemaphoreType.DMA,
    ],
)


def _count_body(dstf_hbm, out_hbm, dstblk, onesb, zb1, acc1):
    cid = lax.axis_index("c")
    sid = lax.axis_index("s")
    wid = sid * NC + cid
    pltpu.sync_copy(dstf_hbm.at[wid], dstblk)
    onev = jnp.ones((L,), _f32)
    zv = jnp.zeros((L,), _f32)
    for j in range(G // L):
        onesb[pl.ds(j * L, L)] = onev
    for j in range(G // L):
        zb1[pl.ds(j * L, L)] = zv

    @pl.when(sid < 8)
    def _():
        z0 = sid * (CACC // 8)
        for j in range(CACC // 8 // G):
            pltpu.sync_copy(zb1, acc1.at[pl.ds(z0 + j * G, G)])

    plsc.subcore_barrier()

    def cbatch(g, _):
        pltpu.sync_copy(onesb, acc1.at[dstblk.at[g]], add=True)
        return 0

    lax.fori_loop(0, NBF, cbatch, 0)
    plsc.subcore_barrier()

    @pl.when(sid == 0)
    def _():
        pltpu.sync_copy(acc1, out_hbm.at[cid])


_count = pl.kernel(
    _count_body,
    out_type=jax.ShapeDtypeStruct((NC, CACC), _f32),
    mesh=_MESH,
    scratch_types=[
        pltpu.VMEM((NBF, G), _i32),
        pltpu.VMEM((G,), _f32),
        pltpu.VMEM((G,), _f32),
        pltpu.VMEM_SHARED((CACC,), _f32),
    ],
)


def _decoder_body(us_hbm, ut_hbm, row3_hbm, col3_hbm, w2_hbm, b2_hbm, out_hbm,
                  rowblk, colblk, usr, utr, outblk, w2b, b2b, sem1, sem2):
    cid = lax.axis_index("c")
    sid = lax.axis_index("s")
    wid = sid * NC + cid
    base = wid * EW
    iota = lax.iota(_i32, L)
    zv = jnp.zeros((L,), _f32)
    pltpu.sync_copy(row3_hbm.at[wid], rowblk)
    pltpu.sync_copy(col3_hbm.at[wid], colblk)
    pltpu.sync_copy(w2_hbm, w2b)
    pltpu.sync_copy(b2_hbm, b2b)
    w2v = [w2b[pl.ds(g * L, L)] for g in range(D // L)]
    b2v = b2b[pl.ds(0, L)]
    last = jnp.full((L,), L - 1, _i32)

    def batch(bi, _):
        cp1 = pltpu.async_copy(us_hbm.at[rowblk.at[bi]], usr, sem1)
        cp2 = pltpu.async_copy(ut_hbm.at[colblk.at[bi]], utr, sem2)
        cp1.wait()
        cp2.wait()

        def sub(s16, _):
            e0 = s16 * L

            def edge(j, ov):
                e = e0 + j
                accv = zv
                for g in range(D // L):
                    a = usr[e, pl.ds(g * L, L)]
                    b = utr[e, pl.ds(g * L, L)]
                    accv = accv + jnp.maximum(a + b, 0.0) * w2v[g]
                x = accv
                for k in (1, 2, 4, 8):
                    sh = jnp.take(x, jnp.maximum(iota - k, 0))
                    x = x + jnp.where(iota >= k, sh, zv)
                tot = jnp.take(x, last)
                return jnp.where(iota == j, tot, ov)

            ov = lax.fori_loop(0, L, edge, zv)
            outblk[pl.ds(bi * G + e0, L)] = ov + b2v
            return 0

        lax.fori_loop(0, G // L, sub, 0)
        return 0

    lax.fori_loop(0, NBD, batch, 0)
    pltpu.sync_copy(outblk, out_hbm.at[pl.ds(base, EW)])


_decoder_sc = pl.kernel(
    _decoder_body,
    out_type=jax.ShapeDtypeStruct((ELP,), _f32),
    mesh=_MESH,
    scratch_types=[
        pltpu.VMEM((NBD, G), _i32),
        pltpu.VMEM((NBD, G), _i32),
        pltpu.VMEM((G, D), _f32),
        pltpu.VMEM((G, D), _f32),
        pltpu.VMEM((EW,), _f32),
        pltpu.VMEM((D,), _f32),
        pltpu.VMEM((D,), _f32),
        pltpu.SemaphoreType.DMA,
        pltpu.SemaphoreType.DMA,
    ],
)


def _sage1_block(agg_ref, inv_ref, x_ref, wl_ref, wr_ref, b_ref, o_ref):
    mean = agg_ref[...] * inv_ref[...]
    acc = jnp.dot(mean, wl_ref[...], preferred_element_type=_f32)
    acc += jnp.dot(x_ref[...], wr_ref[...], preferred_element_type=_f32)
    o_ref[...] = jnp.maximum(acc + b_ref[...], 0.0)


def _sage1(agg, inv, x_dst, Wl, Wr, b, block=1000):
    n = x_dst.shape[0]
    return pl.pallas_call(
        _sage1_block,
        grid=(n // block,),
        in_specs=[
            pl.BlockSpec((block, D), lambda i: (i, 0)),
            pl.BlockSpec((block, 1), lambda i: (i, 0)),
            pl.BlockSpec((block, D), lambda i: (i, 0)),
            pl.BlockSpec((D, D), lambda i: (0, 0)),
            pl.BlockSpec((D, D), lambda i: (0, 0)),
            pl.BlockSpec((1, D), lambda i: (0, 0)),
        ],
        out_specs=pl.BlockSpec((block, D), lambda i: (i, 0)),
        out_shape=jax.ShapeDtypeStruct((n, D), _f32),
    )(agg, inv, x_dst, Wl, Wr, b.reshape(1, D))


def _sage2_block(agg_ref, inv_ref, x_ref, wl_ref, wr_ref, b_ref, wc_ref,
                 bc_ref, o_ref):
    mean = agg_ref[...] * inv_ref[...]
    acc = jnp.dot(mean, wl_ref[...], preferred_element_type=_f32)
    acc += jnp.dot(x_ref[...], wr_ref[...], preferred_element_type=_f32)
    t = jnp.maximum(acc + b_ref[...], 0.0)
    o_ref[...] = (
        jnp.dot(t, wc_ref[...], preferred_element_type=_f32) + bc_ref[...]
    )


def _sage2(agg, inv, x_dst, Wl, Wr, b, Wc, bc, block=1000):
    n = x_dst.shape[0]
    return pl.pallas_call(
        _sage2_block,
        grid=(n // block,),
        in_specs=[
            pl.BlockSpec((block, D), lambda i: (i, 0)),
            pl.BlockSpec((block, 1), lambda i: (i, 0)),
            pl.BlockSpec((block, D), lambda i: (i, 0)),
            pl.BlockSpec((D, D), lambda i: (0, 0)),
            pl.BlockSpec((D, D), lambda i: (0, 0)),
            pl.BlockSpec((1, D), lambda i: (0, 0)),
            pl.BlockSpec((D, D), lambda i: (0, 0)),
            pl.BlockSpec((1, D), lambda i: (0, 0)),
        ],
        out_specs=pl.BlockSpec((block, D), lambda i: (i, 0)),
        out_shape=jax.ShapeDtypeStruct((n, D), _f32),
    )(agg, inv, x_dst, Wl, Wr, b.reshape(1, D), Wc, bc.reshape(1, D))


def _prep_edges(ei):
    src, dst = ei[0], ei[1]
    npad = EP - E_EDGES
    ar = jnp.arange(npad, dtype=_i32)
    dstf = jnp.concatenate([dst, N_NODES + (ar % (CACC - N_NODES))])
    qid = dst // CH
    rank = jnp.zeros((E_EDGES,), _i32)
    counts = []
    for q in range(NCHUNK):
        m = qid == q
        r = jnp.cumsum(m.astype(_i32))
        rank = jnp.where(m, r - 1, rank)
        counts.append(r[-1])
    counts = jnp.stack(counts)
    padded = ((counts + (BKAL - 1)) // BKAL) * BKAL
    soff = jnp.concatenate(
        [jnp.zeros((1,), _i32), jnp.cumsum(padded)[:-1].astype(_i32)])
    nbat = padded // G
    slot = soff[qid] + rank
    dfill = CH + (jnp.arange(EPAD, dtype=_i32) % (ACCR - CH))
    base = jnp.stack([jnp.zeros((EPAD,), _i32), dfill], axis=1)
    pay = jnp.stack([src, dst - qid * CH], axis=1)
    packed = base.at[slot].set(pay, unique_indices=True)
    srcp = packed[:, 0]
    dstp = packed[:, 1]
    sboff = soff // (8 * G)
    nsb = padded // (8 * G)
    z5 = jnp.zeros((5,), _i32)
    mrow = jnp.stack([
        jnp.concatenate([sboff[c::NC], z5, nsb[c::NC].astype(_i32), z5])
        for c in range(NC)])
    meta = jnp.zeros((NC, 8, G), _i32).at[:, 0, :16].set(mrow)
    return (srcp.reshape(EPAD // (8 * G), 8, G),
            dstp.reshape(EPAD // (8 * G), 8, G), meta,
            dstf.reshape(NW, NBF, G))


def _seg_full(x, srcp, dstp, meta, tok=None):
    if tok is not None:
        meta = meta + tok
    o = _segsum(x, srcp, dstp, meta)
    agg = o.reshape(NCHUNK * CH, D)[:N_NODES]
    return agg, (agg[0, 0] * 0.0).astype(_i32)


def kernel(x_sotu, x_taxon, edge_index_st, edge_index_ts, edge_label_index,
           Wl1_st, bl1_st, Wr1_st, Wl1_ts, bl1_ts, Wr1_ts,
           Wl2_st, bl2_st, Wr2_st, Wl2_ts, bl2_ts, Wr2_ts,
           Wlin_s, blin_s, Wlin_t, blin_t, W1, b1, W2, b2):
    src_st, dst_st, meta_st, dstf_st = _prep_edges(edge_index_st)
    src_ts, dst_ts, meta_ts, dstf_ts = _prep_edges(edge_index_ts)

    cpair_t = _count(dstf_st)
    tokc = (cpair_t[0, 0] * 0.0).astype(_i32)
    cpair_s = _count(dstf_ts + tokc)
    cnt_t = cpair_t[0, :N_NODES] + cpair_t[1, :N_NODES]
    cnt_s = cpair_s[0, :N_NODES] + cpair_s[1, :N_NODES]
    inv_t = (1.0 / jnp.maximum(cnt_t, 1.0))[:, None]
    inv_s = (1.0 / jnp.maximum(cnt_s, 1.0))[:, None]

    tk0 = (cpair_s[0, 0] * 0.0).astype(_i32)
    agg1_t, tk = _seg_full(x_sotu, src_st, dst_st, meta_st, tok=tk0)
    agg1_s, _ = _seg_full(x_taxon, src_ts, dst_ts, meta_ts, tok=tk)

    h1_t = _sage1(agg1_t, inv_t, x_taxon, Wl1_st, Wr1_st, bl1_st)
    h1_s = _sage1(agg1_s, inv_s, x_sotu, Wl1_ts, Wr1_ts, bl1_ts)

    agg2_t, tk2 = _seg_full(h1_s, src_st, dst_st, meta_st)
    agg2_s, _ = _seg_full(h1_t, src_ts, dst_ts, meta_ts, tok=tk2)

    W1a, W1b = W1[:D], W1[D:]
    Wc_s = Wlin_s @ W1a
    bc_s = blin_s @ W1a + b1
    Wc_t = Wlin_t @ W1b
    bc_t = blin_t @ W1b
    u_t = _sage2(agg2_t, inv_t, h1_t, Wl2_st, Wr2_st, bl2_st, Wc_t, bc_t)
    u_s = _sage2(agg2_s, inv_s, h1_s, Wl2_ts, Wr2_ts, bl2_ts, Wc_s, bc_s)

    npad = ELP - EL
    spread = (jnp.arange(npad, dtype=_i32) * 101) % N_NODES
    row3 = jnp.concatenate([edge_label_index[0], spread]).reshape(NW, NBD, G)
    col3 = jnp.concatenate([edge_label_index[1], spread]).reshape(NW, NBD, G)
    w2 = W2.reshape(D)
    b2v = jnp.full((D,), b2[0], _f32)
    out_pad = _decoder_sc(u_s, u_t, row3, col3, w2, b2v)
    return out_pad[:EL]


# speedup vs baseline: 1.2343x; 1.2343x over previous
import jax
import jax.numpy as jnp
from jax import lax
from jax.experimental import pallas as pl
from jax.experimental.pallas import tpu as pltpu
from jax.experimental.pallas import tpu_sc as plsc

D = 128
NC, NS, L = 2, 16, 16
NW = NC * NS
N_NODES = 50000
E_EDGES = 500000
G = 128
NCHUNK = 6
CH = 8448
ACCR = 8576
CPT = CH // NS
ZPT = ACCR // NS
BKAL = 2048
EPAD = 516096
EP = 524288
NBF = 128
SHF = NBF * G
CACC = 50176

EL = 200000
NBD = 56
EW = NBD * G
ELP = NW * EW

_MESH = plsc.VectorSubcoreMesh(
    core_axis_name="c", subcore_axis_name="s", num_cores=NC, num_subcores=NS
)

_f32 = jnp.float32
_i32 = jnp.int32


def _segsum_body(x_hbm, src_hbm, dst_hbm, meta_hbm, out_hbm,
                 ss0, dd0, rb0, rb1, metab, zbuf, acc, gs0, gs1):
    cid = lax.axis_index("c")
    sid = lax.axis_index("s")
    zv = jnp.zeros((L,), _f32)
    pltpu.sync_copy(meta_hbm.at[cid], metab)
    mv = metab[0, pl.ds(0, 16)]

    def zr(r, _):
        for j in range(D // L):
            zbuf[r, pl.ds(j * L, L)] = zv
        return 0

    lax.fori_loop(0, G, zr, 0)
    z0 = sid * ZPT
    c0 = sid * CPT
    for qq in range(NCHUNK // NC):
        q = qq * NC + cid
        for j in range(ZPT // G):
            pltpu.sync_copy(zbuf, acc.at[pl.ds(z0 + j * G, G)])
        rem = ZPT % G
        if rem:
            pltpu.sync_copy(zbuf.at[pl.ds(0, rem)],
                            acc.at[pl.ds(z0 + (ZPT // G) * G, rem)])
        plsc.subcore_barrier()
        soff = mv[qq]
        nsb = mv[8 + qq]
        my_nb = (nsb - sid + (NS - 1)) // NS

        def body(k, _):
            sb = soff + sid + k * NS
            pltpu.sync_copy(src_hbm.at[sb], ss0)
            pltpu.sync_copy(dst_hbm.at[sb], dd0)
            pltpu.async_copy(x_hbm.at[ss0.at[0]], rb0, gs0)
            for j in range(1, 8):
                rb_j, gs_j = (rb0, gs0) if j % 2 == 0 else (rb1, gs1)
                rb_p, gs_p = (rb0, gs0) if j % 2 == 1 else (rb1, gs1)
                pltpu.async_copy(x_hbm.at[ss0.at[j]], rb_j, gs_j)
                pltpu.make_async_copy(
                    x_hbm.at[ss0.at[j - 1]], rb_p, gs_p).wait()
                pltpu.sync_copy(rb_p, acc.at[dd0.at[j - 1]], add=True)
            pltpu.make_async_copy(x_hbm.at[ss0.at[7]], rb1, gs1).wait()
            pltpu.sync_copy(rb1, acc.at[dd0.at[7]], add=True)
            return 0

        lax.fori_loop(0, my_nb, body, 0)
        plsc.subcore_barrier()
        pltpu.sync_copy(acc.at[pl.ds(c0, CPT)], out_hbm.at[q, pl.ds(c0, CPT)])
        plsc.subcore_barrier()


_segsum = pl.kernel(
    _segsum_body,
    out_type=jax.ShapeDtypeStruct((NCHUNK, CH, D), _f32),
    mesh=_MESH,
    scratch_types=[
        pltpu.VMEM((8, G), _i32),
        pltpu.VMEM((8, G), _i32),
        pltpu.VMEM((G, D), _f32),
        pltpu.VMEM((G, D), _f32),
        pltpu.VMEM((8, G), _i32),
        pltpu.VMEM((G, D), _f32),
        pltpu.VMEM_SHARED((ACCR, D), _f32),
        pltpu.SemaphoreType.DMA,
        pltpu.SemaphoreType.DMA,
    ],
)


def _count_body(dstf_hbm, out_hbm, dstblk, onesb, zb1, acc1):
    cid = lax.axis_index("c")
    sid = lax.axis_index("s")
    wid = sid * NC + cid
    pltpu.sync_copy(dstf_hbm.at[wid], dstblk)
    onev = jnp.ones((L,), _f32)
    zv = jnp.zeros((L,), _f32)
    for j in range(G // L):
        onesb[pl.ds(j * L, L)] = onev
    for j in range(G // L):
        zb1[pl.ds(j * L, L)] = zv

    @pl.when(sid < 8)
    def _():
        z0 = sid * (CACC // 8)
        for j in range(CACC // 8 // G):
            pltpu.sync_copy(zb1, acc1.at[pl.ds(z0 + j * G, G)])

    plsc.subcore_barrier()

    def cbatch(g, _):
        pltpu.sync_copy(onesb, acc1.at[dstblk.at[g]], add=True)
        return 0

    lax.fori_loop(0, NBF, cbatch, 0)
    plsc.subcore_barrier()

    @pl.when(sid == 0)
    def _():
        pltpu.sync_copy(acc1, out_hbm.at[cid])


_count = pl.kernel(
    _count_body,
    out_type=jax.ShapeDtypeStruct((NC, CACC), _f32),
    mesh=_MESH,
    scratch_types=[
        pltpu.VMEM((NBF, G), _i32),
        pltpu.VMEM((G,), _f32),
        pltpu.VMEM((G,), _f32),
        pltpu.VMEM_SHARED((CACC,), _f32),
    ],
)


def _decoder_body(us_hbm, ut_hbm, row3_hbm, col3_hbm, w2_hbm, b2_hbm, out_hbm,
                  rowblk, colblk, usr, utr, outblk, w2b, b2b, sem1, sem2):
    cid = lax.axis_index("c")
    sid = lax.axis_index("s")
    wid = sid * NC + cid
    base = wid * EW
    iota = lax.iota(_i32, L)
    zv = jnp.zeros((L,), _f32)
    pltpu.sync_copy(row3_hbm.at[wid], rowblk)
    pltpu.sync_copy(col3_hbm.at[wid], colblk)
    pltpu.sync_copy(w2_hbm, w2b)
    pltpu.sync_copy(b2_hbm, b2b)
    w2v = [w2b[pl.ds(g * L, L)] for g in range(D // L)]
    b2v = b2b[pl.ds(0, L)]
    last = jnp.full((L,), L - 1, _i32)

    def batch(bi, _):
        cp1 = pltpu.async_copy(us_hbm.at[rowblk.at[bi]], usr, sem1)
        cp2 = pltpu.async_copy(ut_hbm.at[colblk.at[bi]], utr, sem2)
        cp1.wait()
        cp2.wait()

        def sub(s16, _):
            e0 = s16 * L

            def edge(j, ov):
                e = e0 + j
                accv = zv
                for g in range(D // L):
                    a = usr[e, pl.ds(g * L, L)]
                    b = utr[e, pl.ds(g * L, L)]
                    accv = accv + jnp.maximum(a + b, 0.0) * w2v[g]
                x = accv
                for k in (1, 2, 4, 8):
                    sh = jnp.take(x, jnp.maximum(iota - k, 0))
                    x = x + jnp.where(iota >= k, sh, zv)
                tot = jnp.take(x, last)
                return jnp.where(iota == j, tot, ov)

            ov = lax.fori_loop(0, L, edge, zv)
            outblk[pl.ds(bi * G + e0, L)] = ov + b2v
            return 0

        lax.fori_loop(0, G // L, sub, 0)
        return 0

    lax.fori_loop(0, NBD, batch, 0)
    pltpu.sync_copy(outblk, out_hbm.at[pl.ds(base, EW)])


_decoder_sc = pl.kernel(
    _decoder_body,
    out_type=jax.ShapeDtypeStruct((ELP,), _f32),
    mesh=_MESH,
    scratch_types=[
        pltpu.VMEM((NBD, G), _i32),
        pltpu.VMEM((NBD, G), _i32),
        pltpu.VMEM((G, D), _f32),
        pltpu.VMEM((G, D), _f32),
        pltpu.VMEM((EW,), _f32),
        pltpu.VMEM((D,), _f32),
        pltpu.VMEM((D,), _f32),
        pltpu.SemaphoreType.DMA,
        pltpu.SemaphoreType.DMA,
    ],
)


def _sage1_block(agg_ref, inv_ref, x_ref, wl_ref, wr_ref, b_ref, o_ref):
    mean = agg_ref[...] * inv_ref[...]
    acc = jnp.dot(mean, wl_ref[...], preferred_element_type=_f32)
    acc += jnp.dot(x_ref[...], wr_ref[...], preferred_element_type=_f32)
    o_ref[...] = jnp.maximum(acc + b_ref[...], 0.0)


def _sage1(agg, inv, x_dst, Wl, Wr, b, block=1000):
    n = x_dst.shape[0]
    return pl.pallas_call(
        _sage1_block,
        grid=(n // block,),
        in_specs=[
            pl.BlockSpec((block, D), lambda i: (i, 0)),
            pl.BlockSpec((block, 1), lambda i: (i, 0)),
            pl.BlockSpec((block, D), lambda i: (i, 0)),
            pl.BlockSpec((D, D), lambda i: (0, 0)),
            pl.BlockSpec((D, D), lambda i: (0, 0)),
            pl.BlockSpec((1, D), lambda i: (0, 0)),
        ],
        out_specs=pl.BlockSpec((block, D), lambda i: (i, 0)),
        out_shape=jax.ShapeDtypeStruct((n, D), _f32),
    )(agg, inv, x_dst, Wl, Wr, b.reshape(1, D))


def _sage2_block(agg_ref, inv_ref, x_ref, wl_ref, wr_ref, b_ref, wc_ref,
                 bc_ref, o_ref):
    mean = agg_ref[...] * inv_ref[...]
    acc = jnp.dot(mean, wl_ref[...], preferred_element_type=_f32)
    acc += jnp.dot(x_ref[...], wr_ref[...], preferred_element_type=_f32)
    t = jnp.maximum(acc + b_ref[...], 0.0)
    o_ref[...] = (
        jnp.dot(t, wc_ref[...], preferred_element_type=_f32) + bc_ref[...]
    )


def _sage2(agg, inv, x_dst, Wl, Wr, b, Wc, bc, block=1000):
    n = x_dst.shape[0]
    return pl.pallas_call(
        _sage2_block,
        grid=(n // block,),
        in_specs=[
            pl.BlockSpec((block, D), lambda i: (i, 0)),
            pl.BlockSpec((block, 1), lambda i: (i, 0)),
            pl.BlockSpec((block, D), lambda i: (i, 0)),
            pl.BlockSpec((D, D), lambda i: (0, 0)),
            pl.BlockSpec((D, D), lambda i: (0, 0)),
            pl.BlockSpec((1, D), lambda i: (0, 0)),
            pl.BlockSpec((D, D), lambda i: (0, 0)),
            pl.BlockSpec((1, D), lambda i: (0, 0)),
        ],
        out_specs=pl.BlockSpec((block, D), lambda i: (i, 0)),
        out_shape=jax.ShapeDtypeStruct((n, D), _f32),
    )(agg, inv, x_dst, Wl, Wr, b.reshape(1, D), Wc, bc.reshape(1, D))


def _prep_edges(ei):
    src, dst = ei[0], ei[1]
    npad = EP - E_EDGES
    ar = jnp.arange(npad, dtype=_i32)
    dstf = jnp.concatenate([dst, N_NODES + (ar % (CACC - N_NODES))])
    qid = dst // CH
    rank = jnp.zeros((E_EDGES,), _i32)
    counts = []
    for q in range(NCHUNK):
        m = qid == q
        r = jnp.cumsum(m.astype(_i32))
        rank = jnp.where(m, r - 1, rank)
        counts.append(r[-1])
    counts = jnp.stack(counts)
    padded = ((counts + (BKAL - 1)) // BKAL) * BKAL
    soff = jnp.concatenate(
        [jnp.zeros((1,), _i32), jnp.cumsum(padded)[:-1].astype(_i32)])
    nbat = padded // G
    slot = soff[qid] + rank
    srcp = jnp.zeros((EPAD,), _i32).at[slot].set(src, unique_indices=True)
    dfill = CH + (jnp.arange(EPAD, dtype=_i32) % (ACCR - CH))
    dstp = dfill.at[slot].set(dst - qid * CH, unique_indices=True)
    sboff = soff // (8 * G)
    nsb = padded // (8 * G)
    z5 = jnp.zeros((5,), _i32)
    mrow = jnp.stack([
        jnp.concatenate([sboff[c::NC], z5, nsb[c::NC].astype(_i32), z5])
        for c in range(NC)])
    meta = jnp.zeros((NC, 8, G), _i32).at[:, 0, :16].set(mrow)
    return (srcp.reshape(EPAD // (8 * G), 8, G),
            dstp.reshape(EPAD // (8 * G), 8, G), meta,
            dstf.reshape(NW, NBF, G))


def _seg_full(x, srcp, dstp, meta, tok=None):
    if tok is not None:
        meta = meta + tok
    o = _segsum(x, srcp, dstp, meta)
    agg = o.reshape(NCHUNK * CH, D)[:N_NODES]
    return agg, (agg[0, 0] * 0.0).astype(_i32)


def kernel(x_sotu, x_taxon, edge_index_st, edge_index_ts, edge_label_index,
           Wl1_st, bl1_st, Wr1_st, Wl1_ts, bl1_ts, Wr1_ts,
           Wl2_st, bl2_st, Wr2_st, Wl2_ts, bl2_ts, Wr2_ts,
           Wlin_s, blin_s, Wlin_t, blin_t, W1, b1, W2, b2):
    src_st, dst_st, meta_st, dstf_st = _prep_edges(edge_index_st)
    src_ts, dst_ts, meta_ts, dstf_ts = _prep_edges(edge_index_ts)

    cpair_t = _count(dstf_st)
    tokc = (cpair_t[0, 0] * 0.0).astype(_i32)
    cpair_s = _count(dstf_ts + tokc)
    cnt_t = cpair_t[0, :N_NODES] + cpair_t[1, :N_NODES]
    cnt_s = cpair_s[0, :N_NODES] + cpair_s[1, :N_NODES]
    inv_t = (1.0 / jnp.maximum(cnt_t, 1.0))[:, None]
    inv_s = (1.0 / jnp.maximum(cnt_s, 1.0))[:, None]

    tk0 = (cpair_s[0, 0] * 0.0).astype(_i32)
    agg1_t, tk = _seg_full(x_sotu, src_st, dst_st, meta_st, tok=tk0)
    agg1_s, _ = _seg_full(x_taxon, src_ts, dst_ts, meta_ts, tok=tk)

    h1_t = _sage1(agg1_t, inv_t, x_taxon, Wl1_st, Wr1_st, bl1_st)
    h1_s = _sage1(agg1_s, inv_s, x_sotu, Wl1_ts, Wr1_ts, bl1_ts)

    agg2_t, tk2 = _seg_full(h1_s, src_st, dst_st, meta_st)
    agg2_s, _ = _seg_full(h1_t, src_ts, dst_ts, meta_ts, tok=tk2)

    W1a, W1b = W1[:D], W1[D:]
    Wc_s = Wlin_s @ W1a
    bc_s = blin_s @ W1a + b1
    Wc_t = Wlin_t @ W1b
    bc_t = blin_t @ W1b
    u_t = _sage2(agg2_t, inv_t, h1_t, Wl2_st, Wr2_st, bl2_st, Wc_t, bc_t)
    u_s = _sage2(agg2_s, inv_s, h1_s, Wl2_ts, Wr2_ts, bl2_ts, Wc_s, bc_s)

    npad = ELP - EL
    spread = (jnp.arange(npad, dtype=_i32) * 101) % N_NODES
    row3 = jnp.concatenate([edge_label_index[0], spread]).reshape(NW, NBD, G)
    col3 = jnp.concatenate([edge_label_index[1], spread]).reshape(NW, NBD, G)
    w2 = W2.reshape(D)
    b2v = jnp.full((D,), b2[0], _f32)
    out_pad = _decoder_sc(u_s, u_t, row3, col3, w2, b2v)
    return out_pad[:EL]


# speedup vs baseline: 1.2354x; 1.0009x over previous
import jax
import jax.numpy as jnp
from jax import lax
from jax.experimental import pallas as pl
from jax.experimental.pallas import tpu as pltpu
from jax.experimental.pallas import tpu_sc as plsc

D = 128
NC, NS, L = 2, 16, 16
NW = NC * NS
N_NODES = 50000
E_EDGES = 500000
G = 128
NCHUNK = 6
CH = 8448
ACCR = 8576
CPT = CH // NS
ZPT = ACCR // NS
BKAL = 2048
EPAD = 516096
EP = 524288
NBF = 128
SHF = NBF * G
CACC = 50176

EL = 200000
NBD = 56
EW = NBD * G
ELP = NW * EW

_MESH = plsc.VectorSubcoreMesh(
    core_axis_name="c", subcore_axis_name="s", num_cores=NC, num_subcores=NS
)

_f32 = jnp.float32
_i32 = jnp.int32


def _segsum_body(x_hbm, src_hbm, dst_hbm, meta_hbm, out_hbm,
                 ss0, dd0, rb0, rb1, metab, zbuf, acc, gs0, gs1):
    cid = lax.axis_index("c")
    sid = lax.axis_index("s")
    zv = jnp.zeros((L,), _f32)
    pltpu.sync_copy(meta_hbm.at[cid], metab)
    mv = metab[0, pl.ds(0, 16)]

    def zr(r, _):
        for j in range(D // L):
            zbuf[r, pl.ds(j * L, L)] = zv
        return 0

    lax.fori_loop(0, G, zr, 0)
    z0 = sid * ZPT
    c0 = sid * CPT
    for qq in range(NCHUNK // NC):
        q = qq * NC + cid
        for j in range(ZPT // G):
            pltpu.sync_copy(zbuf, acc.at[pl.ds(z0 + j * G, G)])
        rem = ZPT % G
        if rem:
            pltpu.sync_copy(zbuf.at[pl.ds(0, rem)],
                            acc.at[pl.ds(z0 + (ZPT // G) * G, rem)])
        plsc.subcore_barrier()
        soff = mv[qq]
        nsb = mv[8 + qq]
        my_nb = (nsb - sid + (NS - 1)) // NS

        def body(k, _):
            sb = soff + sid + k * NS
            pltpu.sync_copy(src_hbm.at[sb], ss0)
            pltpu.sync_copy(dst_hbm.at[sb], dd0)
            pltpu.async_copy(x_hbm.at[ss0.at[0]], rb0, gs0)
            for j in range(1, 8):
                rb_j, gs_j = (rb0, gs0) if j % 2 == 0 else (rb1, gs1)
                rb_p, gs_p = (rb0, gs0) if j % 2 == 1 else (rb1, gs1)
                pltpu.async_copy(x_hbm.at[ss0.at[j]], rb_j, gs_j)
                pltpu.make_async_copy(
                    x_hbm.at[ss0.at[j - 1]], rb_p, gs_p).wait()
                pltpu.sync_copy(rb_p, acc.at[dd0.at[j - 1]], add=True)
            pltpu.make_async_copy(x_hbm.at[ss0.at[7]], rb1, gs1).wait()
            pltpu.sync_copy(rb1, acc.at[dd0.at[7]], add=True)
            return 0

        lax.fori_loop(0, my_nb, body, 0)
        plsc.subcore_barrier()
        pltpu.sync_copy(acc.at[pl.ds(c0, CPT)], out_hbm.at[q, pl.ds(c0, CPT)])
        plsc.subcore_barrier()


_segsum = pl.kernel(
    _segsum_body,
    out_type=jax.ShapeDtypeStruct((NCHUNK, CH, D), _f32),
    mesh=_MESH,
    scratch_types=[
        pltpu.VMEM((8, G), _i32),
        pltpu.VMEM((8, G), _i32),
        pltpu.VMEM((G, D), _f32),
        pltpu.VMEM((G, D), _f32),
        pltpu.VMEM((8, G), _i32),
        pltpu.VMEM((G, D), _f32),
        pltpu.VMEM_SHARED((ACCR, D), _f32),
        pltpu.SemaphoreType.DMA,
        pltpu.SemaphoreType.DMA,
    ],
)


def _count_body(dstf_hbm, out_hbm, dstblk, onesb, zb1, acc1):
    cid = lax.axis_index("c")
    sid = lax.axis_index("s")
    wid = sid * NC + cid
    pltpu.sync_copy(dstf_hbm.at[wid], dstblk)
    onev = jnp.ones((L,), _f32)
    zv = jnp.zeros((L,), _f32)
    for j in range(G // L):
        onesb[pl.ds(j * L, L)] = onev
    for j in range(G // L):
        zb1[pl.ds(j * L, L)] = zv

    @pl.when(sid < 8)
    def _():
        z0 = sid * (CACC // 8)
        for j in range(CACC // 8 // G):
            pltpu.sync_copy(zb1, acc1.at[pl.ds(z0 + j * G, G)])

    plsc.subcore_barrier()

    def cbatch(g, _):
        pltpu.sync_copy(onesb, acc1.at[dstblk.at[g]], add=True)
        return 0

    lax.fori_loop(0, NBF, cbatch, 0)
    plsc.subcore_barrier()

    @pl.when(sid == 0)
    def _():
        pltpu.sync_copy(acc1, out_hbm.at[cid])


_count = pl.kernel(
    _count_body,
    out_type=jax.ShapeDtypeStruct((NC, CACC), _f32),
    mesh=_MESH,
    scratch_types=[
        pltpu.VMEM((NBF, G), _i32),
        pltpu.VMEM((G,), _f32),
        pltpu.VMEM((G,), _f32),
        pltpu.VMEM_SHARED((CACC,), _f32),
    ],
)


def _decoder_body(us_hbm, ut_hbm, row3_hbm, col3_hbm, w2_hbm, b2_hbm, out_hbm,
                  rowblk, colblk, usr, utr, outblk, w2b, b2b, sem1, sem2):
    cid = lax.axis_index("c")
    sid = lax.axis_index("s")
    wid = sid * NC + cid
    base = wid * EW
    iota = lax.iota(_i32, L)
    zv = jnp.zeros((L,), _f32)
    pltpu.sync_copy(row3_hbm.at[wid], rowblk)
    pltpu.sync_copy(col3_hbm.at[wid], colblk)
    pltpu.sync_copy(w2_hbm, w2b)
    pltpu.sync_copy(b2_hbm, b2b)
    w2v = [w2b[pl.ds(g * L, L)] for g in range(D // L)]
    b2v = b2b[pl.ds(0, L)]
    last = jnp.full((L,), L - 1, _i32)

    def batch(bi, _):
        cp1 = pltpu.async_copy(us_hbm.at[rowblk.at[bi]], usr, sem1)
        cp2 = pltpu.async_copy(ut_hbm.at[colblk.at[bi]], utr, sem2)
        cp1.wait()
        cp2.wait()

        def sub(s16, _):
            e0 = s16 * L

            def edge(j, ov):
                e = e0 + j
                accv = zv
                for g in range(D // L):
                    a = usr[e, pl.ds(g * L, L)]
                    b = utr[e, pl.ds(g * L, L)]
                    accv = accv + jnp.maximum(a + b, 0.0) * w2v[g]
                x = accv
                for k in (1, 2, 4, 8):
                    sh = jnp.take(x, jnp.maximum(iota - k, 0))
                    x = x + jnp.where(iota >= k, sh, zv)
                tot = jnp.take(x, last)
                return jnp.where(iota == j, tot, ov)

            ov = lax.fori_loop(0, L, edge, zv)
            outblk[pl.ds(bi * G + e0, L)] = ov + b2v
            return 0

        lax.fori_loop(0, G // L, sub, 0)
        return 0

    lax.fori_loop(0, NBD, batch, 0)
    pltpu.sync_copy(outblk, out_hbm.at[pl.ds(base, EW)])


_decoder_sc = pl.kernel(
    _decoder_body,
    out_type=jax.ShapeDtypeStruct((ELP,), _f32),
    mesh=_MESH,
    scratch_types=[
        pltpu.VMEM((NBD, G), _i32),
        pltpu.VMEM((NBD, G), _i32),
        pltpu.VMEM((G, D), _f32),
        pltpu.VMEM((G, D), _f32),
        pltpu.VMEM((EW,), _f32),
        pltpu.VMEM((D,), _f32),
        pltpu.VMEM((D,), _f32),
        pltpu.SemaphoreType.DMA,
        pltpu.SemaphoreType.DMA,
    ],
)


def _sage1_block(agg_ref, inv_ref, x_ref, wl_ref, wr_ref, b_ref, o_ref):
    mean = agg_ref[...] * inv_ref[...]
    acc = jnp.dot(mean, wl_ref[...], preferred_element_type=_f32)
    acc += jnp.dot(x_ref[...], wr_ref[...], preferred_element_type=_f32)
    o_ref[...] = jnp.maximum(acc + b_ref[...], 0.0)


def _sage1(agg, inv, x_dst, Wl, Wr, b, block=1000):
    n = x_dst.shape[0]
    return pl.pallas_call(
        _sage1_block,
        grid=(n // block,),
        in_specs=[
            pl.BlockSpec((block, D), lambda i: (i, 0)),
            pl.BlockSpec((block, 1), lambda i: (i, 0)),
            pl.BlockSpec((block, D), lambda i: (i, 0)),
            pl.BlockSpec((D, D), lambda i: (0, 0)),
            pl.BlockSpec((D, D), lambda i: (0, 0)),
            pl.BlockSpec((1, D), lambda i: (0, 0)),
        ],
        out_specs=pl.BlockSpec((block, D), lambda i: (i, 0)),
        out_shape=jax.ShapeDtypeStruct((n, D), _f32),
    )(agg, inv, x_dst, Wl, Wr, b.reshape(1, D))


def _sage2_block(agg_ref, inv_ref, x_ref, wl_ref, wr_ref, b_ref, wc_ref,
                 bc_ref, o_ref):
    mean = agg_ref[...] * inv_ref[...]
    acc = jnp.dot(mean, wl_ref[...], preferred_element_type=_f32)
    acc += jnp.dot(x_ref[...], wr_ref[...], preferred_element_type=_f32)
    t = jnp.maximum(acc + b_ref[...], 0.0)
    o_ref[...] = (
        jnp.dot(t, wc_ref[...], preferred_element_type=_f32) + bc_ref[...]
    )


def _sage2(agg, inv, x_dst, Wl, Wr, b, Wc, bc, block=1000):
    n = x_dst.shape[0]
    return pl.pallas_call(
        _sage2_block,
        grid=(n // block,),
        in_specs=[
            pl.BlockSpec((block, D), lambda i: (i, 0)),
            pl.BlockSpec((block, 1), lambda i: (i, 0)),
            pl.BlockSpec((block, D), lambda i: (i, 0)),
            pl.BlockSpec((D, D), lambda i: (0, 0)),
            pl.BlockSpec((D, D), lambda i: (0, 0)),
            pl.BlockSpec((1, D), lambda i: (0, 0)),
            pl.BlockSpec((D, D), lambda i: (0, 0)),
            pl.BlockSpec((1, D), lambda i: (0, 0)),
        ],
        out_specs=pl.BlockSpec((block, D), lambda i: (i, 0)),
        out_shape=jax.ShapeDtypeStruct((n, D), _f32),
    )(agg, inv, x_dst, Wl, Wr, b.reshape(1, D), Wc, bc.reshape(1, D))


def _prep_edges(ei):
    src, dst = ei[0], ei[1]
    npad = EP - E_EDGES
    ar = jnp.arange(npad, dtype=_i32)
    dstf = jnp.concatenate([dst, N_NODES + (ar % (CACC - N_NODES))])
    qid = dst // CH
    RR, CC = 500, 1000
    m3 = (qid.reshape(RR, CC)[None, :, :]
          == jnp.arange(NCHUNK, dtype=_i32)[:, None, None]).astype(_f32)
    tri = (jnp.arange(CC)[:, None] <= jnp.arange(CC)[None, :]).astype(_f32)
    intra = jnp.einsum("qrc,cd->qrd", m3, tri)
    rowsum = intra[:, :, -1]
    stri = (jnp.arange(RR)[:, None] < jnp.arange(RR)[None, :]).astype(_f32)
    rowoff = jnp.einsum("qr,rs->qs", rowsum, stri)
    counts = rowsum.sum(axis=1).astype(_i32)
    sel = (m3 * (intra + rowoff[:, :, None])).sum(axis=0)
    rank = (sel - 1.0).astype(_i32).reshape(E_EDGES)
    padded = ((counts + (BKAL - 1)) // BKAL) * BKAL
    soff = jnp.concatenate(
        [jnp.zeros((1,), _i32), jnp.cumsum(padded)[:-1].astype(_i32)])
    nbat = padded // G
    slot = soff[qid] + rank
    srcp = jnp.zeros((EPAD,), _i32).at[slot].set(src, unique_indices=True)
    dfill = CH + (jnp.arange(EPAD, dtype=_i32) % (ACCR - CH))
    dstp = dfill.at[slot].set(dst - qid * CH, unique_indices=True)
    sboff = soff // (8 * G)
    nsb = padded // (8 * G)
    z5 = jnp.zeros((5,), _i32)
    mrow = jnp.stack([
        jnp.concatenate([sboff[c::NC], z5, nsb[c::NC].astype(_i32), z5])
        for c in range(NC)])
    meta = jnp.zeros((NC, 8, G), _i32).at[:, 0, :16].set(mrow)
    return (srcp.reshape(EPAD // (8 * G), 8, G),
            dstp.reshape(EPAD // (8 * G), 8, G), meta,
            dstf.reshape(NW, NBF, G))


def _seg_full(x, srcp, dstp, meta, tok=None):
    if tok is not None:
        meta = meta + tok
    o = _segsum(x, srcp, dstp, meta)
    agg = o.reshape(NCHUNK * CH, D)[:N_NODES]
    return agg, (agg[0, 0] * 0.0).astype(_i32)


def kernel(x_sotu, x_taxon, edge_index_st, edge_index_ts, edge_label_index,
           Wl1_st, bl1_st, Wr1_st, Wl1_ts, bl1_ts, Wr1_ts,
           Wl2_st, bl2_st, Wr2_st, Wl2_ts, bl2_ts, Wr2_ts,
           Wlin_s, blin_s, Wlin_t, blin_t, W1, b1, W2, b2):
    src_st, dst_st, meta_st, dstf_st = _prep_edges(edge_index_st)
    src_ts, dst_ts, meta_ts, dstf_ts = _prep_edges(edge_index_ts)

    cpair_t = _count(dstf_st)
    tokc = (cpair_t[0, 0] * 0.0).astype(_i32)
    cpair_s = _count(dstf_ts + tokc)
    cnt_t = cpair_t[0, :N_NODES] + cpair_t[1, :N_NODES]
    cnt_s = cpair_s[0, :N_NODES] + cpair_s[1, :N_NODES]
    inv_t = (1.0 / jnp.maximum(cnt_t, 1.0))[:, None]
    inv_s = (1.0 / jnp.maximum(cnt_s, 1.0))[:, None]

    tk0 = (cpair_s[0, 0] * 0.0).astype(_i32)
    agg1_t, tk = _seg_full(x_sotu, src_st, dst_st, meta_st, tok=tk0)
    agg1_s, _ = _seg_full(x_taxon, src_ts, dst_ts, meta_ts, tok=tk)

    h1_t = _sage1(agg1_t, inv_t, x_taxon, Wl1_st, Wr1_st, bl1_st)
    h1_s = _sage1(agg1_s, inv_s, x_sotu, Wl1_ts, Wr1_ts, bl1_ts)

    agg2_t, tk2 = _seg_full(h1_s, src_st, dst_st, meta_st)
    agg2_s, _ = _seg_full(h1_t, src_ts, dst_ts, meta_ts, tok=tk2)

    W1a, W1b = W1[:D], W1[D:]
    Wc_s = Wlin_s @ W1a
    bc_s = blin_s @ W1a + b1
    Wc_t = Wlin_t @ W1b
    bc_t = blin_t @ W1b
    u_t = _sage2(agg2_t, inv_t, h1_t, Wl2_st, Wr2_st, bl2_st, Wc_t, bc_t)
    u_s = _sage2(agg2_s, inv_s, h1_s, Wl2_ts, Wr2_ts, bl2_ts, Wc_s, bc_s)

    npad = ELP - EL
    spread = (jnp.arange(npad, dtype=_i32) * 101) % N_NODES
    row3 = jnp.concatenate([edge_label_index[0], spread]).reshape(NW, NBD, G)
    col3 = jnp.concatenate([edge_label_index[1], spread]).reshape(NW, NBD, G)
    w2 = W2.reshape(D)
    b2v = jnp.full((D,), b2[0], _f32)
    out_pad = _decoder_sc(u_s, u_t, row3, col3, w2, b2v)
    return out_pad[:EL]


# speedup vs baseline: 4.6887x; 3.7952x over previous
import jax
import jax.numpy as jnp
from jax import lax
from jax.experimental import pallas as pl
from jax.experimental.pallas import tpu as pltpu
from jax.experimental.pallas import tpu_sc as plsc

D = 128
NC, NS, L = 2, 16, 16
NW = NC * NS
N_NODES = 50000
E_EDGES = 500000
G = 128
NCHUNK = 6
CH = 8448
ACCR = 8576
CPT = CH // NS
ZPT = ACCR // NS
BKAL = 2048
EPAD = 516096
EP = 524288
NBF = 128
SHF = NBF * G
CACC = 50176
SLOTW = 518144
SPW = SLOTW // NS
NSBR = EP // NS // (8 * G)

EL = 200000
NBD = 56
EW = NBD * G
ELP = NW * EW

_MESH = plsc.VectorSubcoreMesh(
    core_axis_name="c", subcore_axis_name="s", num_cores=NC, num_subcores=NS
)

_f32 = jnp.float32
_i32 = jnp.int32


def _segsum_body(x_hbm, src_hbm, dst_hbm, meta_hbm, out_hbm,
                 ss0, dd0, rb0, rb1, metab, zbuf, acc, gs0, gs1):
    cid = lax.axis_index("c")
    sid = lax.axis_index("s")
    zv = jnp.zeros((L,), _f32)
    pltpu.sync_copy(meta_hbm.at[cid], metab)
    mv = metab[0, pl.ds(0, 16)]

    def zr(r, _):
        for j in range(D // L):
            zbuf[r, pl.ds(j * L, L)] = zv
        return 0

    lax.fori_loop(0, G, zr, 0)
    z0 = sid * ZPT
    c0 = sid * CPT
    for qq in range(NCHUNK // NC):
        q = qq * NC + cid
        for j in range(ZPT // G):
            pltpu.sync_copy(zbuf, acc.at[pl.ds(z0 + j * G, G)])
        rem = ZPT % G
        if rem:
            pltpu.sync_copy(zbuf.at[pl.ds(0, rem)],
                            acc.at[pl.ds(z0 + (ZPT // G) * G, rem)])
        plsc.subcore_barrier()
        soff = mv[qq]
        nsb = mv[8 + qq]
        my_nb = (nsb - sid + (NS - 1)) // NS

        def body(k, _):
            sb = soff + sid + k * NS
            pltpu.sync_copy(src_hbm.at[sb], ss0)
            pltpu.sync_copy(dst_hbm.at[sb], dd0)
            pltpu.async_copy(x_hbm.at[ss0.at[0]], rb0, gs0)
            for j in range(1, 8):
                rb_j, gs_j = (rb0, gs0) if j % 2 == 0 else (rb1, gs1)
                rb_p, gs_p = (rb0, gs0) if j % 2 == 1 else (rb1, gs1)
                pltpu.async_copy(x_hbm.at[ss0.at[j]], rb_j, gs_j)
                pltpu.make_async_copy(
                    x_hbm.at[ss0.at[j - 1]], rb_p, gs_p).wait()
                pltpu.sync_copy(rb_p, acc.at[dd0.at[j - 1]], add=True)
            pltpu.make_async_copy(x_hbm.at[ss0.at[7]], rb1, gs1).wait()
            pltpu.sync_copy(rb1, acc.at[dd0.at[7]], add=True)
            return 0

        lax.fori_loop(0, my_nb, body, 0)
        plsc.subcore_barrier()
        pltpu.sync_copy(acc.at[pl.ds(c0, CPT)], out_hbm.at[q, pl.ds(c0, CPT)])
        plsc.subcore_barrier()


_segsum = pl.kernel(
    _segsum_body,
    out_type=jax.ShapeDtypeStruct((NCHUNK, CH, D), _f32),
    mesh=_MESH,
    scratch_types=[
        pltpu.VMEM((8, G), _i32),
        pltpu.VMEM((8, G), _i32),
        pltpu.VMEM((G, D), _f32),
        pltpu.VMEM((G, D), _f32),
        pltpu.VMEM((8, G), _i32),
        pltpu.VMEM((G, D), _f32),
        pltpu.VMEM_SHARED((ACCR, D), _f32),
        pltpu.SemaphoreType.DMA,
        pltpu.SemaphoreType.DMA,
    ],
)


def _route_body(slot_hbm, val_hbm, out_hbm, slotb, valb, pbuf, accs):
    cid = lax.axis_index("c")
    sid = lax.axis_index("s")
    iota = lax.iota(_i32, L)
    for k in range(G // L):
        v = jnp.where(cid == 1, iota + (CH + k * L), 0)
        pbuf[pl.ds(k * L, L)] = v
    base = sid * SPW
    for j in range(SPW // G):
        pltpu.sync_copy(pbuf, accs.at[pl.ds(base + j * G, G)])
    plsc.subcore_barrier()

    def sbody(sb, _):
        pltpu.sync_copy(slot_hbm.at[sid, sb], slotb)
        pltpu.sync_copy(val_hbm.at[cid, sid, sb], valb)
        for j in range(8):
            pltpu.sync_copy(valb.at[j], accs.at[slotb.at[j]])
        return 0

    lax.fori_loop(0, NSBR, sbody, 0)
    plsc.subcore_barrier()
    pltpu.sync_copy(accs.at[pl.ds(base, SPW)],
                    out_hbm.at[cid, pl.ds(base, SPW)])


_route = pl.kernel(
    _route_body,
    out_type=jax.ShapeDtypeStruct((NC, SLOTW), _i32),
    mesh=_MESH,
    scratch_types=[
        pltpu.VMEM((8, G), _i32),
        pltpu.VMEM((8, G), _i32),
        pltpu.VMEM((G,), _i32),
        pltpu.VMEM_SHARED((SLOTW,), _i32),
    ],
)


def _count_body(dstf_hbm, out_hbm, dstblk, onesb, zb1, acc1):
    cid = lax.axis_index("c")
    sid = lax.axis_index("s")
    wid = sid * NC + cid
    pltpu.sync_copy(dstf_hbm.at[wid], dstblk)
    onev = jnp.ones((L,), _f32)
    zv = jnp.zeros((L,), _f32)
    for j in range(G // L):
        onesb[pl.ds(j * L, L)] = onev
    for j in range(G // L):
        zb1[pl.ds(j * L, L)] = zv

    @pl.when(sid < 8)
    def _():
        z0 = sid * (CACC // 8)
        for j in range(CACC // 8 // G):
            pltpu.sync_copy(zb1, acc1.at[pl.ds(z0 + j * G, G)])

    plsc.subcore_barrier()

    def cbatch(g, _):
        pltpu.sync_copy(onesb, acc1.at[dstblk.at[g]], add=True)
        return 0

    lax.fori_loop(0, NBF, cbatch, 0)
    plsc.subcore_barrier()

    @pl.when(sid == 0)
    def _():
        pltpu.sync_copy(acc1, out_hbm.at[cid])


_count = pl.kernel(
    _count_body,
    out_type=jax.ShapeDtypeStruct((NC, CACC), _f32),
    mesh=_MESH,
    scratch_types=[
        pltpu.VMEM((NBF, G), _i32),
        pltpu.VMEM((G,), _f32),
        pltpu.VMEM((G,), _f32),
        pltpu.VMEM_SHARED((CACC,), _f32),
    ],
)


def _decoder_body(us_hbm, ut_hbm, row3_hbm, col3_hbm, w2_hbm, b2_hbm, out_hbm,
                  rowblk, colblk, usr, utr, outblk, w2b, b2b, sem1, sem2):
    cid = lax.axis_index("c")
    sid = lax.axis_index("s")
    wid = sid * NC + cid
    base = wid * EW
    iota = lax.iota(_i32, L)
    zv = jnp.zeros((L,), _f32)
    pltpu.sync_copy(row3_hbm.at[wid], rowblk)
    pltpu.sync_copy(col3_hbm.at[wid], colblk)
    pltpu.sync_copy(w2_hbm, w2b)
    pltpu.sync_copy(b2_hbm, b2b)
    w2v = [w2b[pl.ds(g * L, L)] for g in range(D // L)]
    b2v = b2b[pl.ds(0, L)]
    last = jnp.full((L,), L - 1, _i32)

    def batch(bi, _):
        cp1 = pltpu.async_copy(us_hbm.at[rowblk.at[bi]], usr, sem1)
        cp2 = pltpu.async_copy(ut_hbm.at[colblk.at[bi]], utr, sem2)
        cp1.wait()
        cp2.wait()

        def sub(s16, _):
            e0 = s16 * L

            def edge(j, ov):
                e = e0 + j
                accv = zv
                for g in range(D // L):
                    a = usr[e, pl.ds(g * L, L)]
                    b = utr[e, pl.ds(g * L, L)]
                    accv = accv + jnp.maximum(a + b, 0.0) * w2v[g]
                x = accv
                for k in (1, 2, 4, 8):
                    sh = jnp.take(x, jnp.maximum(iota - k, 0))
                    x = x + jnp.where(iota >= k, sh, zv)
                tot = jnp.take(x, last)
                return jnp.where(iota == j, tot, ov)

            ov = lax.fori_loop(0, L, edge, zv)
            outblk[pl.ds(bi * G + e0, L)] = ov + b2v
            return 0

        lax.fori_loop(0, G // L, sub, 0)
        return 0

    lax.fori_loop(0, NBD, batch, 0)
    pltpu.sync_copy(outblk, out_hbm.at[pl.ds(base, EW)])


_decoder_sc = pl.kernel(
    _decoder_body,
    out_type=jax.ShapeDtypeStruct((ELP,), _f32),
    mesh=_MESH,
    scratch_types=[
        pltpu.VMEM((NBD, G), _i32),
        pltpu.VMEM((NBD, G), _i32),
        pltpu.VMEM((G, D), _f32),
        pltpu.VMEM((G, D), _f32),
        pltpu.VMEM((EW,), _f32),
        pltpu.VMEM((D,), _f32),
        pltpu.VMEM((D,), _f32),
        pltpu.SemaphoreType.DMA,
        pltpu.SemaphoreType.DMA,
    ],
)


def _sage1_block(agg_ref, inv_ref, x_ref, wl_ref, wr_ref, b_ref, o_ref):
    mean = agg_ref[...] * inv_ref[...]
    acc = jnp.dot(mean, wl_ref[...], preferred_element_type=_f32)
    acc += jnp.dot(x_ref[...], wr_ref[...], preferred_element_type=_f32)
    o_ref[...] = jnp.maximum(acc + b_ref[...], 0.0)


def _sage1(agg, inv, x_dst, Wl, Wr, b, block=1000):
    n = x_dst.shape[0]
    return pl.pallas_call(
        _sage1_block,
        grid=(n // block,),
        in_specs=[
            pl.BlockSpec((block, D), lambda i: (i, 0)),
            pl.BlockSpec((block, 1), lambda i: (i, 0)),
            pl.BlockSpec((block, D), lambda i: (i, 0)),
            pl.BlockSpec((D, D), lambda i: (0, 0)),
            pl.BlockSpec((D, D), lambda i: (0, 0)),
            pl.BlockSpec((1, D), lambda i: (0, 0)),
        ],
        out_specs=pl.BlockSpec((block, D), lambda i: (i, 0)),
        out_shape=jax.ShapeDtypeStruct((n, D), _f32),
    )(agg, inv, x_dst, Wl, Wr, b.reshape(1, D))


def _sage2_block(agg_ref, inv_ref, x_ref, wl_ref, wr_ref, b_ref, wc_ref,
                 bc_ref, o_ref):
    mean = agg_ref[...] * inv_ref[...]
    acc = jnp.dot(mean, wl_ref[...], preferred_element_type=_f32)
    acc += jnp.dot(x_ref[...], wr_ref[...], preferred_element_type=_f32)
    t = jnp.maximum(acc + b_ref[...], 0.0)
    o_ref[...] = (
        jnp.dot(t, wc_ref[...], preferred_element_type=_f32) + bc_ref[...]
    )


def _sage2(agg, inv, x_dst, Wl, Wr, b, Wc, bc, block=1000):
    n = x_dst.shape[0]
    return pl.pallas_call(
        _sage2_block,
        grid=(n // block,),
        in_specs=[
            pl.BlockSpec((block, D), lambda i: (i, 0)),
            pl.BlockSpec((block, 1), lambda i: (i, 0)),
            pl.BlockSpec((block, D), lambda i: (i, 0)),
            pl.BlockSpec((D, D), lambda i: (0, 0)),
            pl.BlockSpec((D, D), lambda i: (0, 0)),
            pl.BlockSpec((1, D), lambda i: (0, 0)),
            pl.BlockSpec((D, D), lambda i: (0, 0)),
            pl.BlockSpec((1, D), lambda i: (0, 0)),
        ],
        out_specs=pl.BlockSpec((block, D), lambda i: (i, 0)),
        out_shape=jax.ShapeDtypeStruct((n, D), _f32),
    )(agg, inv, x_dst, Wl, Wr, b.reshape(1, D), Wc, bc.reshape(1, D))


def _prep_edges(ei, tok=None):
    src, dst = ei[0], ei[1]
    if tok is not None:
        src = src + tok
    npad = EP - E_EDGES
    ar = jnp.arange(npad, dtype=_i32)
    dstf = jnp.concatenate([dst, N_NODES + (ar % (CACC - N_NODES))])
    qid = dst // CH
    RR, CC = 500, 1000
    m3 = (qid.reshape(RR, CC)[None, :, :]
          == jnp.arange(NCHUNK, dtype=_i32)[:, None, None]).astype(_f32)
    tri = (jnp.arange(CC)[:, None] <= jnp.arange(CC)[None, :]).astype(_f32)
    intra = jnp.einsum("qrc,cd->qrd", m3, tri)
    rowsum = intra[:, :, -1]
    stri = (jnp.arange(RR)[:, None] < jnp.arange(RR)[None, :]).astype(_f32)
    rowoff = jnp.einsum("qr,rs->qs", rowsum, stri)
    counts = rowsum.sum(axis=1).astype(_i32)
    sel = (m3 * (intra + rowoff[:, :, None])).sum(axis=0)
    rank = (sel - 1.0).astype(_i32).reshape(E_EDGES)
    padded = ((counts + (BKAL - 1)) // BKAL) * BKAL
    soff = jnp.concatenate(
        [jnp.zeros((1,), _i32), jnp.cumsum(padded)[:-1].astype(_i32)])
    nbat = padded // G
    slot = soff[qid] + rank
    npad2 = EP - E_EDGES
    ar2 = jnp.arange(npad2, dtype=_i32)
    slot_p = jnp.concatenate([slot, EPAD + (ar2 % (SLOTW - EPAD))])
    zpad = jnp.zeros((npad2,), _i32)
    vals = jnp.stack([jnp.concatenate([src, zpad]),
                      jnp.concatenate([dst - qid * CH, zpad])])
    ro = _route(slot_p.reshape(NS, NSBR, 8, G),
                vals.reshape(NC, NS, NSBR, 8, G))
    srcp = ro[0, :EPAD]
    dstp = ro[1, :EPAD]
    sboff = soff // (8 * G)
    nsb = padded // (8 * G)
    z5 = jnp.zeros((5,), _i32)
    mrow = jnp.stack([
        jnp.concatenate([sboff[c::NC], z5, nsb[c::NC].astype(_i32), z5])
        for c in range(NC)])
    meta = jnp.zeros((NC, 8, G), _i32).at[:, 0, :16].set(mrow)
    return (srcp.reshape(EPAD // (8 * G), 8, G),
            dstp.reshape(EPAD // (8 * G), 8, G), meta,
            dstf.reshape(NW, NBF, G), (srcp[0] * 0).astype(_i32))


def _seg_full(x, srcp, dstp, meta, tok=None):
    if tok is not None:
        meta = meta + tok
    o = _segsum(x, srcp, dstp, meta)
    agg = o.reshape(NCHUNK * CH, D)[:N_NODES]
    return agg, (agg[0, 0] * 0.0).astype(_i32)


def kernel(x_sotu, x_taxon, edge_index_st, edge_index_ts, edge_label_index,
           Wl1_st, bl1_st, Wr1_st, Wl1_ts, bl1_ts, Wr1_ts,
           Wl2_st, bl2_st, Wr2_st, Wl2_ts, bl2_ts, Wr2_ts,
           Wlin_s, blin_s, Wlin_t, blin_t, W1, b1, W2, b2):
    src_st, dst_st, meta_st, dstf_st, tkr = _prep_edges(edge_index_st)
    src_ts, dst_ts, meta_ts, dstf_ts, tkr2 = _prep_edges(edge_index_ts,
                                                         tok=tkr)

    cpair_t = _count(dstf_st + tkr2)
    tokc = (cpair_t[0, 0] * 0.0).astype(_i32)
    cpair_s = _count(dstf_ts + tokc)
    cnt_t = cpair_t[0, :N_NODES] + cpair_t[1, :N_NODES]
    cnt_s = cpair_s[0, :N_NODES] + cpair_s[1, :N_NODES]
    inv_t = (1.0 / jnp.maximum(cnt_t, 1.0))[:, None]
    inv_s = (1.0 / jnp.maximum(cnt_s, 1.0))[:, None]

    tk0 = (cpair_s[0, 0] * 0.0).astype(_i32)
    agg1_t, tk = _seg_full(x_sotu, src_st, dst_st, meta_st, tok=tk0)
    agg1_s, _ = _seg_full(x_taxon, src_ts, dst_ts, meta_ts, tok=tk)

    h1_t = _sage1(agg1_t, inv_t, x_taxon, Wl1_st, Wr1_st, bl1_st)
    h1_s = _sage1(agg1_s, inv_s, x_sotu, Wl1_ts, Wr1_ts, bl1_ts)

    agg2_t, tk2 = _seg_full(h1_s, src_st, dst_st, meta_st)
    agg2_s, _ = _seg_full(h1_t, src_ts, dst_ts, meta_ts, tok=tk2)

    W1a, W1b = W1[:D], W1[D:]
    Wc_s = Wlin_s @ W1a
    bc_s = blin_s @ W1a + b1
    Wc_t = Wlin_t @ W1b
    bc_t = blin_t @ W1b
    u_t = _sage2(agg2_t, inv_t, h1_t, Wl2_st, Wr2_st, bl2_st, Wc_t, bc_t)
    u_s = _sage2(agg2_s, inv_s, h1_s, Wl2_ts, Wr2_ts, bl2_ts, Wc_s, bc_s)

    npad = ELP - EL
    spread = (jnp.arange(npad, dtype=_i32) * 101) % N_NODES
    row3 = jnp.concatenate([edge_label_index[0], spread]).reshape(NW, NBD, G)
    col3 = jnp.concatenate([edge_label_index[1], spread]).reshape(NW, NBD, G)
    w2 = W2.reshape(D)
    b2v = jnp.full((D,), b2[0], _f32)
    out_pad = _decoder_sc(u_s, u_t, row3, col3, w2, b2v)
    return out_pad[:EL]
